# baseline (speedup 1.0000x reference)
import functools

import jax
import jax.numpy as jnp
from jax import lax
from jax.experimental import pallas as pl
from jax.experimental.pallas import tpu as pltpu
from jax.experimental.pallas import tpu_sc as plsc

R = 128
N = 32768
L = 16
NCHUNK = N // L
NC, NS = 2, 16
NW = NC * NS
ROWS_PER_W = R // NW
MAX_ITERS = 24
G = 64
GSUB = G // L
NG = N // G
SG = 256
SGC = SG // L
NSG = N // SG
UNROLL = 8

_GATHER_DNUMS = lax.GatherDimensionNumbers(
    offset_dims=(), collapsed_slice_dims=(0,), start_index_map=(0,))


def _gather16(x, idx):
    return lax.gather(x, idx[:, None], dimension_numbers=_GATHER_DNUMS,
                      slice_sizes=(1,),
                      mode=lax.GatherScatterMode.PROMISE_IN_BOUNDS)


def _lane0(x):
    return lax.squeeze(lax.slice(x, (0,), (1,)), (0,))


def _butterfly(x, op):
    idx = lax.iota(jnp.int32, L)
    for s in (8, 4, 2, 1):
        x = op(x, _gather16(x, jnp.bitwise_xor(idx, s)))
    return x


def _find_tau(row_buf, smax_buf, tau_buf, sm_sup, sm_cand, sm_done):
    zero = jnp.zeros((L,), jnp.float32)
    neg = jnp.full((L,), -3e38, jnp.float32)

    def max_body(sgi, acc):
        base = sgi * SG
        subs = [row_buf[pl.ds(base + u * L, L)] for u in range(4)]
        for u in range(4, SGC):
            subs[u % 4] = jnp.maximum(subs[u % 4],
                                      row_buf[pl.ds(base + u * L, L)])
        sm = jnp.maximum(jnp.maximum(subs[0], subs[1]),
                         jnp.maximum(subs[2], subs[3]))
        smax_buf[pl.ds(sgi * L, L)] = sm
        return jnp.maximum(acc, sm)

    with jax.named_scope("p1a_max"):
        acc = lax.fori_loop(0, NSG, max_body, neg)
    mv = _butterfly(acc, jnp.maximum)
    tau0v = mv - 1.0
    tau0s = _lane0(tau0v)

    def sup_body(i, cnt):
        for w in range(2):
            sgi = i * 2 + w
            a = smax_buf[pl.ds(sgi * L, L)]
            smax = _lane0(_butterfly(a, jnp.maximum))
            sm_sup[cnt] = sgi
            cnt = cnt + (smax > tau0s).astype(jnp.int32)
        return cnt

    def g2_body(j, cnt):
        gbase = sm_sup[j] * (SG // G)
        for w in range(SG // G):
            g = gbase + w
            base = g * G
            a = row_buf[pl.ds(base, L)]
            for u in range(1, GSUB):
                a = jnp.maximum(a, row_buf[pl.ds(base + u * L, L)])
            gmax = _lane0(_butterfly(a, jnp.maximum))
            sm_cand[cnt] = g
            cnt = cnt + (gmax > tau0s).astype(jnp.int32)
        return cnt

    with jax.named_scope("p1b_cand"):
        nsup = lax.fori_loop(0, NSG // 2, sup_body, jnp.int32(0))
        ncand = lax.fori_loop(0, nsup, g2_body, jnp.int32(0))

    tau_buf[pl.ds(0, L)] = tau0v
    sm_done[0] = jnp.int32(0)

    def it_body(t, carry):
        @pl.when(sm_done[0] == 0)
        def _():
            tauv = tau_buf[pl.ds(0, L)]

            def p_body(j, sk):
                s_acc, k_acc = sk
                base = sm_cand[j] * G
                for u in range(GSUB):
                    v = row_buf[pl.ds(base + u * L, L)]
                    mask = v > tauv
                    s_acc = s_acc + jnp.where(mask, v, 0.0)
                    k_acc = k_acc + jnp.where(mask, 1.0, 0.0)
                return (s_acc, k_acc)

            s_acc, k_acc = lax.fori_loop(0, ncand, p_body, (zero, zero))
            sv = _butterfly(s_acc, jnp.add)
            kv = _butterfly(k_acc, jnp.add)
            new_tauv = (sv - 1.0) / kv
            sm_done[0] = (_lane0(new_tauv) <= _lane0(tauv)).astype(jnp.int32)
            tau_buf[pl.ds(0, L)] = jnp.maximum(new_tauv, tauv)
        return carry

    with jax.named_scope("p2_michelot"):
        lax.fori_loop(0, MAX_ITERS, it_body, jnp.int32(0))
    return ncand


def _make_sc_kernel():
    mesh = plsc.VectorSubcoreMesh(core_axis_name="c", subcore_axis_name="s")

    @functools.partial(
        pl.kernel,
        mesh=mesh,
        out_type=jax.ShapeDtypeStruct((R, N), jnp.float32),
        scratch_types=[pltpu.VMEM((N,), jnp.float32),
                       pltpu.VMEM((N,), jnp.float32),
                       pltpu.VMEM((N,), jnp.float32),
                       pltpu.VMEM((NSG * L,), jnp.float32),
                       pltpu.VMEM((L,), jnp.float32),
                       pltpu.SMEM((NSG,), jnp.int32),
                       pltpu.SMEM((NG,), jnp.int32),
                       pltpu.SMEM((NG,), jnp.int32),
                       pltpu.SMEM((1,), jnp.int32),
                       pltpu.SemaphoreType.DMA,
                       pltpu.SemaphoreType.DMA,
                       pltpu.SemaphoreType.DMA],
    )
    def sc_sparsemax(x_hbm, out_hbm, a0, a1, zbuf, smax_buf, tau_buf,
                     sm_sup, sm_cand_a, sm_cand_b, sm_done, si0, si1, so):
        inb, sin = (a0, a1), (si0, si1)
        cands = (sm_cand_a, sm_cand_b)
        wid = lax.axis_index("s") * NC + lax.axis_index("c")
        base_row = wid * ROWS_PER_W
        zero = jnp.zeros((L,), jnp.float32)

        h_in = [None, None]
        h_out = None
        h_in[0] = pltpu.async_copy(x_hbm.at[base_row], inb[0], sin[0])

        def z_body(i, c):
            base = i * (L * UNROLL)
            for u in range(UNROLL):
                zbuf[pl.ds(base + u * L, L)] = zero
            return c

        lax.fori_loop(0, NCHUNK // UNROLL, z_body, jnp.int32(0))

        prev_ncand = jnp.int32(0)
        for r in range(ROWS_PER_W):
            b = r % 2
            if r + 1 < ROWS_PER_W:
                nb = (r + 1) % 2
                h_in[nb] = pltpu.async_copy(
                    x_hbm.at[base_row + r + 1], inb[nb], sin[nb])
            h_in[b].wait()
            rb = inb[b]
            sm_cand = cands[r % 2]
            sm_prev = cands[(r + 1) % 2]
            ncand = _find_tau(rb, smax_buf, tau_buf, sm_sup, sm_cand, sm_done)
            if h_out is not None:
                h_out.wait()

            def rz_body(j, c):
                gb = sm_prev[j] * G
                for u in range(GSUB):
                    zbuf[pl.ds(gb + u * L, L)] = zero
                return c

            lax.fori_loop(0, prev_ncand, rz_body, jnp.int32(0))
            tauv = tau_buf[pl.ds(0, L)]

            def wr_body(j, c):
                gb = sm_cand[j] * G
                for u in range(GSUB):
                    sl = pl.ds(gb + u * L, L)
                    zbuf[sl] = jnp.maximum(rb[sl] - tauv, 0.0)
                return c

            with jax.named_scope("p3_write"):
                lax.fori_loop(0, ncand, wr_body, jnp.int32(0))
            h_out = pltpu.async_copy(zbuf, out_hbm.at[base_row + r], so)
            prev_ncand = ncand
        h_out.wait()

    return sc_sparsemax


_SC_SPARSEMAX = _make_sc_kernel()


def kernel(logits):
    return _SC_SPARSEMAX(logits)


# speedup vs baseline: 1.1198x; 1.1198x over previous
import functools

import jax
import jax.numpy as jnp
from jax import lax
from jax.experimental import pallas as pl
from jax.experimental.pallas import tpu as pltpu
from jax.experimental.pallas import tpu_sc as plsc

R = 128
N = 32768
L = 16
NCHUNK = N // L
NC, NS = 2, 16
NW = NC * NS
ROWS_PER_W = R // NW
MAX_ITERS = 24
G = 64
GSUB = G // L
NG = N // G
SG = 256
SGC = SG // L
NSG = N // SG
UNROLL = 8

_GATHER_DNUMS = lax.GatherDimensionNumbers(
    offset_dims=(), collapsed_slice_dims=(0,), start_index_map=(0,))


def _gather16(x, idx):
    return lax.gather(x, idx[:, None], dimension_numbers=_GATHER_DNUMS,
                      slice_sizes=(1,),
                      mode=lax.GatherScatterMode.PROMISE_IN_BOUNDS)


def _lane0(x):
    return lax.squeeze(lax.slice(x, (0,), (1,)), (0,))


def _butterfly(x, op):
    idx = lax.iota(jnp.int32, L)
    for s in (8, 4, 2, 1):
        x = op(x, _gather16(x, jnp.bitwise_xor(idx, s)))
    return x


def _find_tau(row_buf, tau_buf, sm_cand, sm_done):
    zero = jnp.zeros((L,), jnp.float32)
    neg = jnp.full((L,), -3e38, jnp.float32)

    def max_body(i, accs):
        base = i * (L * UNROLL)
        return tuple(
            jnp.maximum(a, row_buf[pl.ds(base + u * L, L)])
            for u, a in enumerate(accs))

    accs = lax.fori_loop(0, NCHUNK // UNROLL, max_body, (neg,) * UNROLL)
    acc = accs[0]
    for a in accs[1:]:
        acc = jnp.maximum(acc, a)
    mv = _butterfly(acc, jnp.maximum)
    tau0v = mv - 1.0
    tau0s = _lane0(tau0v)

    def cand_body(i, cnt):
        for w in range(2):
            g = i * 2 + w
            base = g * G
            a = row_buf[pl.ds(base, L)]
            for u in range(1, GSUB):
                a = jnp.maximum(a, row_buf[pl.ds(base + u * L, L)])
            gmax = _lane0(_butterfly(a, jnp.maximum))
            sm_cand[cnt] = g
            cnt = cnt + (gmax > tau0s).astype(jnp.int32)
        return cnt

    ncand = lax.fori_loop(0, NG // 2, cand_body, jnp.int32(0))

    tau_buf[pl.ds(0, L)] = tau0v
    sm_done[0] = jnp.int32(0)

    def it_body(t, carry):
        @pl.when(sm_done[0] == 0)
        def _():
            tauv = tau_buf[pl.ds(0, L)]

            def p_body(j, sk):
                s_acc, k_acc = sk
                base = sm_cand[j] * G
                for u in range(GSUB):
                    v = row_buf[pl.ds(base + u * L, L)]
                    mask = v > tauv
                    s_acc = s_acc + jnp.where(mask, v, 0.0)
                    k_acc = k_acc + jnp.where(mask, 1.0, 0.0)
                return (s_acc, k_acc)

            s_acc, k_acc = lax.fori_loop(0, ncand, p_body, (zero, zero))
            sv = _butterfly(s_acc, jnp.add)
            kv = _butterfly(k_acc, jnp.add)
            new_tauv = (sv - 1.0) / kv
            sm_done[0] = (_lane0(new_tauv) <= _lane0(tauv)).astype(jnp.int32)
            tau_buf[pl.ds(0, L)] = jnp.maximum(new_tauv, tauv)
        return carry

    with jax.named_scope("p2_michelot"):
        lax.fori_loop(0, MAX_ITERS, it_body, jnp.int32(0))
    return ncand


def _make_sc_kernel():
    mesh = plsc.VectorSubcoreMesh(core_axis_name="c", subcore_axis_name="s")

    @functools.partial(
        pl.kernel,
        mesh=mesh,
        out_type=jax.ShapeDtypeStruct((R, N), jnp.float32),
        scratch_types=[pltpu.VMEM((N,), jnp.float32),
                       pltpu.VMEM((N,), jnp.float32),
                       pltpu.VMEM((N,), jnp.float32),
                       pltpu.VMEM((L,), jnp.float32),
                       pltpu.SMEM((NG,), jnp.int32),
                       pltpu.SMEM((NG,), jnp.int32),
                       pltpu.SMEM((1,), jnp.int32),
                       pltpu.SemaphoreType.DMA,
                       pltpu.SemaphoreType.DMA,
                       pltpu.SemaphoreType.DMA],
    )
    def sc_sparsemax(x_hbm, out_hbm, a0, a1, zbuf, tau_buf,
                     sm_cand_a, sm_cand_b, sm_done, si0, si1, so):
        inb, sin = (a0, a1), (si0, si1)
        cands = (sm_cand_a, sm_cand_b)
        wid = lax.axis_index("s") * NC + lax.axis_index("c")
        base_row = wid * ROWS_PER_W
        zero = jnp.zeros((L,), jnp.float32)

        h_in = [None, None]
        h_out = None
        h_in[0] = pltpu.async_copy(x_hbm.at[base_row], inb[0], sin[0])

        def z_body(i, c):
            base = i * (L * UNROLL)
            for u in range(UNROLL):
                zbuf[pl.ds(base + u * L, L)] = zero
            return c

        lax.fori_loop(0, NCHUNK // UNROLL, z_body, jnp.int32(0))

        prev_ncand = jnp.int32(0)
        for r in range(ROWS_PER_W):
            b = r % 2
            if r + 1 < ROWS_PER_W:
                nb = (r + 1) % 2
                h_in[nb] = pltpu.async_copy(
                    x_hbm.at[base_row + r + 1], inb[nb], sin[nb])
            h_in[b].wait()
            rb = inb[b]
            sm_cand = cands[r % 2]
            sm_prev = cands[(r + 1) % 2]
            ncand = _find_tau(rb, tau_buf, sm_cand, sm_done)
            if h_out is not None:
                h_out.wait()

            def rz_body(j, c):
                gb = sm_prev[j] * G
                for u in range(GSUB):
                    zbuf[pl.ds(gb + u * L, L)] = zero
                return c

            lax.fori_loop(0, prev_ncand, rz_body, jnp.int32(0))
            tauv = tau_buf[pl.ds(0, L)]

            def wr_body(j, c):
                gb = sm_cand[j] * G
                for u in range(GSUB):
                    sl = pl.ds(gb + u * L, L)
                    zbuf[sl] = jnp.maximum(rb[sl] - tauv, 0.0)
                return c

            with jax.named_scope("p3_write"):
                lax.fori_loop(0, ncand, wr_body, jnp.int32(0))
            h_out = pltpu.async_copy(zbuf, out_hbm.at[base_row + r], so)
            prev_ncand = ncand
        h_out.wait()

    return sc_sparsemax


_SC_SPARSEMAX = _make_sc_kernel()


def kernel(logits):
    return _SC_SPARSEMAX(logits)
